# baseline (speedup 1.0000x reference)
import jax
import jax.numpy as jnp
import numpy as np
from jax.experimental import pallas as pl
from jax.experimental.pallas import tpu as pltpu

B, L, N, D = 32, 96, 16, 64
E, K = 8, 2
D_FF = 128
LN = L * N
EPS = float(np.finfo(float).eps)
SB = 4
NSTEPS = B // SB


def _compute_sample(xm, sw_ref, sbv_ref, wgx_ref, w1_ref, b1_ref, w2_ref,
                    b2_ref):
    p = jnp.dot(xm, sw_ref[...], preferred_element_type=jnp.float32)
    logits = jnp.dot(wgx_ref[...], p, preferred_element_type=jnp.float32)
    logits = logits + sbv_ref[...]
    iota = jax.lax.broadcasted_iota(jnp.int32, (E, 1), 0)
    m1 = jnp.max(logits, axis=0, keepdims=True)
    i1 = jnp.min(jnp.where(logits == m1, iota, E), axis=0, keepdims=True)
    l2 = jnp.where(iota == i1, -jnp.inf, logits)
    m2 = jnp.max(l2, axis=0, keepdims=True)
    i2 = jnp.min(jnp.where(l2 == m2, iota, E), axis=0, keepdims=True)
    r = jnp.exp(m2 - m1)
    g1 = 1.0 / (1.0 + r)
    g2 = r / (1.0 + r)
    e1 = i1[0, 0]
    e2 = i2[0, 0]

    w1a = w1_ref[pl.ds(e1, 1)][0]
    w1b = w1_ref[pl.ds(e2, 1)][0]
    b1a = b1_ref[pl.ds(e1, 1)][0]
    b1b = b1_ref[pl.ds(e2, 1)][0]
    w2a = w2_ref[pl.ds(e1, 1)][0]
    w2b = w2_ref[pl.ds(e2, 1)][0]
    b2a = b2_ref[pl.ds(e1, 1)][0]
    b2b = b2_ref[pl.ds(e2, 1)][0]

    w1 = jnp.concatenate([w1a, w1b], axis=1)
    bias1 = jnp.concatenate([b1a, b1b], axis=1)
    h = jnp.dot(xm.astype(jnp.bfloat16), w1.astype(jnp.bfloat16),
                preferred_element_type=jnp.float32)
    h = jnp.maximum(h + bias1, 0.0)
    y1 = jnp.dot(h[:, :D_FF], w2a,
                 preferred_element_type=jnp.float32) + b2a
    y2 = jnp.dot(h[:, D_FF:], w2b,
                 preferred_element_type=jnp.float32) + b2b

    t = g1 + g2 * jnp.exp(y2 - y1)
    return y1 + jnp.log(t)


def _in_copy(x_hbm, xbuf, insem, step, slot):
    return pltpu.make_async_copy(
        x_hbm.at[pl.ds(step * SB, SB)], xbuf.at[slot], insem.at[slot])


def _out_copy(o_hbm, obuf, outsem, step, slot):
    return pltpu.make_async_copy(
        obuf.at[slot], o_hbm.at[pl.ds(step * SB, SB)], outsem.at[slot])


def _body(sw_ref, sbv_ref, wgx_ref, w1_ref, b1_ref, w2_ref, b2_ref,
          x_hbm, o_hbm, xbuf, obuf, insem, outsem):
    i = pl.program_id(0)
    slot = jax.lax.rem(i, 2)
    nslot = jax.lax.rem(i + 1, 2)

    @pl.when(i == 0)
    def _():
        _in_copy(x_hbm, xbuf, insem, 0, 0).start()

    @pl.when(i + 1 < NSTEPS)
    def _():
        _in_copy(x_hbm, xbuf, insem, i + 1, nslot).start()

    _in_copy(x_hbm, xbuf, insem, i, slot).wait()

    @pl.when(i >= 2)
    def _():
        _out_copy(o_hbm, obuf, outsem, i - 2, slot).wait()

    for s in range(SB):
        xm = xbuf[slot, s].reshape(LN, D)
        res = _compute_sample(xm, sw_ref, sbv_ref, wgx_ref, w1_ref, b1_ref,
                              w2_ref, b2_ref)
        obuf[slot, s] = res.reshape(L, N, D)

    _out_copy(o_hbm, obuf, outsem, i, slot).start()

    @pl.when(i == NSTEPS - 1)
    def _():
        _out_copy(o_hbm, obuf, outsem, i - 1, nslot).wait()
        _out_copy(o_hbm, obuf, outsem, i, slot).wait()


@jax.jit
def kernel(x, start_w, start_b, w_gate, W1, b1, W2, b2):
    wgx = jnp.repeat(w_gate.T / N, N, axis=1)
    sbv = (start_b[0] * jnp.sum(w_gate, axis=0)).reshape(E, 1)

    out = pl.pallas_call(
        _body,
        grid=(NSTEPS,),
        in_specs=[
            pl.BlockSpec((D, 1), lambda b: (0, 0)),
            pl.BlockSpec((E, 1), lambda b: (0, 0)),
            pl.BlockSpec((E, LN), lambda b: (0, 0)),
            pl.BlockSpec((E, D, D_FF), lambda b: (0, 0, 0)),
            pl.BlockSpec((E, 1, D_FF), lambda b: (0, 0, 0)),
            pl.BlockSpec((E, D_FF, D), lambda b: (0, 0, 0)),
            pl.BlockSpec((E, 1, D), lambda b: (0, 0, 0)),
            pl.BlockSpec(memory_space=pl.ANY),
        ],
        out_specs=pl.BlockSpec(memory_space=pl.ANY),
        out_shape=jax.ShapeDtypeStruct((B, L, N, D), jnp.float32),
        scratch_shapes=[
            pltpu.VMEM((2, SB, L, N, D), jnp.float32),
            pltpu.VMEM((2, SB, L, N, D), jnp.float32),
            pltpu.SemaphoreType.DMA((2,)),
            pltpu.SemaphoreType.DMA((2,)),
        ],
    )(start_w, sbv, wgx, W1, b1.reshape(E, 1, D_FF), W2,
      b2.reshape(E, 1, D), x)

    return out


# speedup vs baseline: 1.0284x; 1.0284x over previous
import jax
import jax.numpy as jnp
import numpy as np
from jax.experimental import pallas as pl
from jax.experimental.pallas import tpu as pltpu

B, L, N, D = 32, 96, 16, 64
E, K = 8, 2
D_FF = 128
LN = L * N
EPS = float(np.finfo(float).eps)


SB = 8


def _body(x_ref, sw_ref, sb_ref, wgx_ref, w1_ref, b1_ref, w2_ref, b2_ref,
          o_ref):
    for s in range(SB):
        xm = x_ref[s].reshape(LN, D)

        p = jnp.dot(xm, sw_ref[...], preferred_element_type=jnp.float32)
        logits = jnp.dot(wgx_ref[...], p, preferred_element_type=jnp.float32)
        logits = logits + sb_ref[...]
        iota = jax.lax.broadcasted_iota(jnp.int32, (E, 1), 0)
        m1 = jnp.max(logits, axis=0, keepdims=True)
        i1 = jnp.min(jnp.where(logits == m1, iota, E), axis=0, keepdims=True)
        l2 = jnp.where(iota == i1, -jnp.inf, logits)
        m2 = jnp.max(l2, axis=0, keepdims=True)
        i2 = jnp.min(jnp.where(l2 == m2, iota, E), axis=0, keepdims=True)
        r = jnp.exp(m2 - m1)
        g1 = 1.0 / (1.0 + r)
        g2 = r / (1.0 + r)
        e1 = i1[0, 0]
        e2 = i2[0, 0]

        w1a = w1_ref[pl.ds(e1, 1)][0]
        w1b = w1_ref[pl.ds(e2, 1)][0]
        b1a = b1_ref[pl.ds(e1, 1)][0]
        b1b = b1_ref[pl.ds(e2, 1)][0]
        w2a = w2_ref[pl.ds(e1, 1)][0]
        w2b = w2_ref[pl.ds(e2, 1)][0]
        b2a = b2_ref[pl.ds(e1, 1)][0]
        b2b = b2_ref[pl.ds(e2, 1)][0]

        w1 = jnp.concatenate([w1a, w1b], axis=1)
        bias1 = jnp.concatenate([b1a, b1b], axis=1)
        h = jnp.dot(xm, w1, preferred_element_type=jnp.float32)
        h = jnp.maximum(h + bias1, 0.0)
        y1 = jnp.dot(h[:, :D_FF], w2a,
                     preferred_element_type=jnp.float32) + b2a
        y2 = jnp.dot(h[:, D_FF:], w2b,
                     preferred_element_type=jnp.float32) + b2b

        acc = g1 * jnp.exp(y1) + g2 * jnp.exp(y2)
        acc = jnp.where(acc == 0.0, EPS, acc)
        o_ref[s] = jnp.log(acc).reshape(L, N, D)


@jax.jit
def kernel(x, start_w, start_b, w_gate, W1, b1, W2, b2):
    wgx = jnp.repeat(w_gate.T / N, N, axis=1)
    sb = start_b.reshape(1, 1)

    out = pl.pallas_call(
        _body,
        grid=(B // SB,),
        in_specs=[
            pl.BlockSpec((SB, L, N, D), lambda b: (b, 0, 0, 0)),
            pl.BlockSpec((D, 1), lambda b: (0, 0)),
            pl.BlockSpec((1, 1), lambda b: (0, 0)),
            pl.BlockSpec((E, LN), lambda b: (0, 0)),
            pl.BlockSpec((E, D, D_FF), lambda b: (0, 0, 0)),
            pl.BlockSpec((E, 1, D_FF), lambda b: (0, 0, 0)),
            pl.BlockSpec((E, D_FF, D), lambda b: (0, 0, 0)),
            pl.BlockSpec((E, 1, D), lambda b: (0, 0, 0)),
        ],
        out_specs=pl.BlockSpec((SB, L, N, D), lambda b: (b, 0, 0, 0)),
        out_shape=jax.ShapeDtypeStruct((B, L, N, D), jnp.float32),
    )(x, start_w, sb, wgx, W1, b1.reshape(E, 1, D_FF), W2,
      b2.reshape(E, 1, D))

    return out


# speedup vs baseline: 1.1157x; 1.0850x over previous
import jax
import jax.numpy as jnp
import numpy as np
from jax.experimental import pallas as pl
from jax.experimental.pallas import tpu as pltpu

B, L, N, D = 32, 96, 16, 64
E, K = 8, 2
D_FF = 128
LN = L * N
EPS = float(np.finfo(float).eps)
SB = 4


def _body(x_ref, sw_ref, sbv_ref, wgx_ref, w1_ref, b1_ref, w2_ref, b2_ref,
          o_ref):
    xall = x_ref[...].reshape(SB * LN, D)
    p = jnp.dot(xall, sw_ref[...], preferred_element_type=jnp.float32)
    cols = [
        jnp.dot(wgx_ref[...], p[s * LN:(s + 1) * LN],
                preferred_element_type=jnp.float32)
        for s in range(SB)
    ]
    logits = jnp.concatenate(cols, axis=1) + sbv_ref[...]
    iota = jax.lax.broadcasted_iota(jnp.int32, (E, SB), 0)
    m1 = jnp.max(logits, axis=0, keepdims=True)
    i1 = jnp.min(jnp.where(logits == m1, iota, E), axis=0, keepdims=True)
    l2 = jnp.where(iota == i1, -jnp.inf, logits)
    m2 = jnp.max(l2, axis=0, keepdims=True)
    i2 = jnp.min(jnp.where(l2 == m2, iota, E), axis=0, keepdims=True)
    r = jnp.exp(m2 - m1)
    g1 = 1.0 / (1.0 + r)
    g2 = r / (1.0 + r)

    for s in range(SB):
        e1 = i1[0, s]
        e2 = i2[0, s]

        w1a = w1_ref[pl.ds(e1, 1)][0]
        w1b = w1_ref[pl.ds(e2, 1)][0]
        b1a = b1_ref[pl.ds(e1, 1)][0]
        b1b = b1_ref[pl.ds(e2, 1)][0]
        w2a = w2_ref[pl.ds(e1, 1)][0]
        w2b = w2_ref[pl.ds(e2, 1)][0]
        b2a = b2_ref[pl.ds(e1, 1)][0]
        b2b = b2_ref[pl.ds(e2, 1)][0]

        xm = x_ref[s].reshape(LN, D)
        w1 = jnp.concatenate([w1a, w1b], axis=1)
        bias1 = jnp.concatenate([b1a, b1b], axis=1)
        h = jnp.dot(xm, w1, preferred_element_type=jnp.float32)
        h = jnp.maximum(h + bias1, 0.0)
        y1 = jnp.dot(h[:, :D_FF], w2a,
                     preferred_element_type=jnp.float32) + b2a
        y2 = jnp.dot(h[:, D_FF:], w2b,
                     preferred_element_type=jnp.float32) + b2b

        acc = g1[:, s:s + 1] * jnp.exp(y1) + g2[:, s:s + 1] * jnp.exp(y2)
        acc = jnp.where(acc == 0.0, EPS, acc)
        o_ref[s] = jnp.log(acc).reshape(L, N, D)


@jax.jit
def kernel(x, start_w, start_b, w_gate, W1, b1, W2, b2):
    wgx = jnp.repeat(w_gate.T / N, N, axis=1)
    sbv = (start_b[0] * jnp.sum(w_gate, axis=0)).reshape(E, 1)

    out = pl.pallas_call(
        _body,
        grid=(B // SB,),
        in_specs=[
            pl.BlockSpec((SB, L, N, D), lambda b: (b, 0, 0, 0)),
            pl.BlockSpec((D, 1), lambda b: (0, 0)),
            pl.BlockSpec((E, 1), lambda b: (0, 0)),
            pl.BlockSpec((E, LN), lambda b: (0, 0)),
            pl.BlockSpec((E, D, D_FF), lambda b: (0, 0, 0)),
            pl.BlockSpec((E, 1, D_FF), lambda b: (0, 0, 0)),
            pl.BlockSpec((E, D_FF, D), lambda b: (0, 0, 0)),
            pl.BlockSpec((E, 1, D), lambda b: (0, 0, 0)),
        ],
        out_specs=pl.BlockSpec((SB, L, N, D), lambda b: (b, 0, 0, 0)),
        out_shape=jax.ShapeDtypeStruct((B, L, N, D), jnp.float32),
    )(x, start_w, sbv, wgx, W1, b1.reshape(E, 1, D_FF), W2,
      b2.reshape(E, 1, D))

    return out


# speedup vs baseline: 1.1214x; 1.0051x over previous
import jax
import jax.numpy as jnp
import numpy as np
from jax.experimental import pallas as pl
from jax.experimental.pallas import tpu as pltpu

B, L, N, D = 32, 96, 16, 64
E, K = 8, 2
D_FF = 128
LN = L * N
EPS = float(np.finfo(float).eps)
SB = 8


def _body(x_ref, sw_ref, sbv_ref, wgx_ref, w1_ref, b1_ref, w2_ref, b2_ref,
          o_ref):
    xall = x_ref[...].reshape(SB * LN, D)
    p = jnp.dot(xall, sw_ref[...], preferred_element_type=jnp.float32)
    cols = [
        jnp.dot(wgx_ref[...], p[s * LN:(s + 1) * LN],
                preferred_element_type=jnp.float32)
        for s in range(SB)
    ]
    logits = jnp.concatenate(cols, axis=1) + sbv_ref[...]
    iota = jax.lax.broadcasted_iota(jnp.int32, (E, SB), 0)
    m1 = jnp.max(logits, axis=0, keepdims=True)
    i1 = jnp.min(jnp.where(logits == m1, iota, E), axis=0, keepdims=True)
    l2 = jnp.where(iota == i1, -jnp.inf, logits)
    m2 = jnp.max(l2, axis=0, keepdims=True)
    i2 = jnp.min(jnp.where(l2 == m2, iota, E), axis=0, keepdims=True)
    r = jnp.exp(m2 - m1)
    g1 = 1.0 / (1.0 + r)
    g2 = r / (1.0 + r)

    for s in range(SB):
        e1 = i1[0, s]
        e2 = i2[0, s]

        w1a = w1_ref[pl.ds(e1, 1)][0]
        w1b = w1_ref[pl.ds(e2, 1)][0]
        b1a = b1_ref[pl.ds(e1, 1)][0]
        b1b = b1_ref[pl.ds(e2, 1)][0]
        w2a = w2_ref[pl.ds(e1, 1)][0]
        w2b = w2_ref[pl.ds(e2, 1)][0]
        b2a = b2_ref[pl.ds(e1, 1)][0]
        b2b = b2_ref[pl.ds(e2, 1)][0]

        xm = x_ref[s].reshape(LN, D)
        w1 = jnp.concatenate([w1a, w1b], axis=1)
        bias1 = jnp.concatenate([b1a, b1b], axis=1)
        h = jnp.dot(xm, w1, preferred_element_type=jnp.float32)
        h = jnp.maximum(h + bias1, 0.0)
        y1 = jnp.dot(h[:, :D_FF], w2a,
                     preferred_element_type=jnp.float32) + b2a
        y2 = jnp.dot(h[:, D_FF:], w2b,
                     preferred_element_type=jnp.float32) + b2b

        acc = g1[:, s:s + 1] * jnp.exp(y1) + g2[:, s:s + 1] * jnp.exp(y2)
        acc = jnp.where(acc == 0.0, EPS, acc)
        o_ref[s] = jnp.log(acc).reshape(L, N, D)


@jax.jit
def kernel(x, start_w, start_b, w_gate, W1, b1, W2, b2):
    wgx = jnp.repeat(w_gate.T / N, N, axis=1)
    sbv = (start_b[0] * jnp.sum(w_gate, axis=0)).reshape(E, 1)

    out = pl.pallas_call(
        _body,
        grid=(B // SB,),
        in_specs=[
            pl.BlockSpec((SB, L, N, D), lambda b: (b, 0, 0, 0)),
            pl.BlockSpec((D, 1), lambda b: (0, 0)),
            pl.BlockSpec((E, 1), lambda b: (0, 0)),
            pl.BlockSpec((E, LN), lambda b: (0, 0)),
            pl.BlockSpec((E, D, D_FF), lambda b: (0, 0, 0)),
            pl.BlockSpec((E, 1, D_FF), lambda b: (0, 0, 0)),
            pl.BlockSpec((E, D_FF, D), lambda b: (0, 0, 0)),
            pl.BlockSpec((E, 1, D), lambda b: (0, 0, 0)),
        ],
        out_specs=pl.BlockSpec((SB, L, N, D), lambda b: (b, 0, 0, 0)),
        out_shape=jax.ShapeDtypeStruct((B, L, N, D), jnp.float32),
    )(x, start_w, sbv, wgx, W1, b1.reshape(E, 1, D_FF), W2,
      b2.reshape(E, 1, D))

    return out


# speedup vs baseline: 1.1349x; 1.0120x over previous
import jax
import jax.numpy as jnp
import numpy as np
from jax.experimental import pallas as pl
from jax.experimental.pallas import tpu as pltpu

B, L, N, D = 32, 96, 16, 64
E, K = 8, 2
D_FF = 128
LN = L * N
EPS = float(np.finfo(float).eps)
SB = 8


def _body(x_ref, sw_ref, sbv_ref, wgx_ref, w1_ref, b1_ref, w2_ref, b2_ref,
          o_ref):
    xall = x_ref[...].reshape(SB * LN, D)
    p = jnp.dot(xall, sw_ref[...], preferred_element_type=jnp.float32)
    cols = [
        jnp.dot(wgx_ref[...], p[s * LN:(s + 1) * LN],
                preferred_element_type=jnp.float32)
        for s in range(SB)
    ]
    logits = jnp.concatenate(cols, axis=1) + sbv_ref[...]
    iota = jax.lax.broadcasted_iota(jnp.int32, (E, SB), 0)
    m1 = jnp.max(logits, axis=0, keepdims=True)
    i1 = jnp.min(jnp.where(logits == m1, iota, E), axis=0, keepdims=True)
    l2 = jnp.where(iota == i1, -jnp.inf, logits)
    m2 = jnp.max(l2, axis=0, keepdims=True)
    i2 = jnp.min(jnp.where(l2 == m2, iota, E), axis=0, keepdims=True)
    r = jnp.exp(m2 - m1)
    g1 = 1.0 / (1.0 + r)
    g2 = r / (1.0 + r)

    for s in range(SB):
        e1 = i1[0, s]
        e2 = i2[0, s]

        w1a = w1_ref[pl.ds(e1, 1)][0]
        w1b = w1_ref[pl.ds(e2, 1)][0]
        b1a = b1_ref[pl.ds(e1, 1)][0]
        b1b = b1_ref[pl.ds(e2, 1)][0]
        w2a = w2_ref[pl.ds(e1, 1)][0]
        w2b = w2_ref[pl.ds(e2, 1)][0]
        b2a = b2_ref[pl.ds(e1, 1)][0]
        b2b = b2_ref[pl.ds(e2, 1)][0]

        xm = x_ref[s].reshape(LN, D)
        w1 = jnp.concatenate([w1a, w1b], axis=1)
        bias1 = jnp.concatenate([b1a, b1b], axis=1)
        h = jnp.dot(xm, w1, preferred_element_type=jnp.float32)
        h = jnp.maximum(h + bias1, 0.0)
        y1 = jnp.dot(h[:, :D_FF], w2a,
                     preferred_element_type=jnp.float32) + b2a
        y2 = jnp.dot(h[:, D_FF:], w2b,
                     preferred_element_type=jnp.float32) + b2b

        ey = jnp.exp(jnp.concatenate([y1, y2], axis=1))
        acc = g1[:, s:s + 1] * ey[:, :D] + g2[:, s:s + 1] * ey[:, D:]
        acc = jnp.where(acc == 0.0, EPS, acc)
        o_ref[s] = jnp.log(acc).reshape(L, N, D)


@jax.jit
def kernel(x, start_w, start_b, w_gate, W1, b1, W2, b2):
    wgx = jnp.repeat(w_gate.T / N, N, axis=1)
    sbv = (start_b[0] * jnp.sum(w_gate, axis=0)).reshape(E, 1)

    out = pl.pallas_call(
        _body,
        grid=(B // SB,),
        in_specs=[
            pl.BlockSpec((SB, L, N, D), lambda b: (b, 0, 0, 0)),
            pl.BlockSpec((D, 1), lambda b: (0, 0)),
            pl.BlockSpec((E, 1), lambda b: (0, 0)),
            pl.BlockSpec((E, LN), lambda b: (0, 0)),
            pl.BlockSpec((E, D, D_FF), lambda b: (0, 0, 0)),
            pl.BlockSpec((E, 1, D_FF), lambda b: (0, 0, 0)),
            pl.BlockSpec((E, D_FF, D), lambda b: (0, 0, 0)),
            pl.BlockSpec((E, 1, D), lambda b: (0, 0, 0)),
        ],
        out_specs=pl.BlockSpec((SB, L, N, D), lambda b: (b, 0, 0, 0)),
        out_shape=jax.ShapeDtypeStruct((B, L, N, D), jnp.float32),
    )(x, start_w, sbv, wgx, W1, b1.reshape(E, 1, D_FF), W2,
      b2.reshape(E, 1, D))

    return out


# speedup vs baseline: 1.1685x; 1.0296x over previous
import jax
import jax.numpy as jnp
import numpy as np
from jax.experimental import pallas as pl
from jax.experimental.pallas import tpu as pltpu

B, L, N, D = 32, 96, 16, 64
E, K = 8, 2
D_FF = 128
LN = L * N
EPS = float(np.finfo(float).eps)
SB = 8


def _body(x_ref, sw_ref, sbv_ref, wgx_ref, w1_ref, b1_ref, w2_ref, b2_ref,
          o_ref):
    xall = x_ref[...].reshape(SB * LN, D)
    p = jnp.dot(xall, sw_ref[...], preferred_element_type=jnp.float32)
    cols = [
        jnp.dot(wgx_ref[...], p[s * LN:(s + 1) * LN],
                preferred_element_type=jnp.float32)
        for s in range(SB)
    ]
    logits = jnp.concatenate(cols, axis=1) + sbv_ref[...]
    iota = jax.lax.broadcasted_iota(jnp.int32, (E, SB), 0)
    m1 = jnp.max(logits, axis=0, keepdims=True)
    i1 = jnp.min(jnp.where(logits == m1, iota, E), axis=0, keepdims=True)
    l2 = jnp.where(iota == i1, -jnp.inf, logits)
    m2 = jnp.max(l2, axis=0, keepdims=True)
    i2 = jnp.min(jnp.where(l2 == m2, iota, E), axis=0, keepdims=True)
    r = jnp.exp(m2 - m1)
    g1 = 1.0 / (1.0 + r)
    g2 = r / (1.0 + r)

    for s in range(SB):
        e1 = i1[0, s]
        e2 = i2[0, s]

        w1a = w1_ref[pl.ds(e1, 1)][0]
        w1b = w1_ref[pl.ds(e2, 1)][0]
        b1a = b1_ref[pl.ds(e1, 1)][0]
        b1b = b1_ref[pl.ds(e2, 1)][0]
        w2a = w2_ref[pl.ds(e1, 1)][0]
        w2b = w2_ref[pl.ds(e2, 1)][0]
        b2a = b2_ref[pl.ds(e1, 1)][0]
        b2b = b2_ref[pl.ds(e2, 1)][0]

        xm = x_ref[s].reshape(LN, D)
        w1 = jnp.concatenate([w1a, w1b], axis=1)
        bias1 = jnp.concatenate([b1a, b1b], axis=1)
        h = jnp.dot(xm, w1, preferred_element_type=jnp.float32)
        h = jnp.maximum(h + bias1, 0.0)
        z = jnp.zeros((D_FF, D), jnp.float32)
        w2d = jnp.concatenate(
            [jnp.concatenate([w2a, z], axis=1),
             jnp.concatenate([z, w2b], axis=1)], axis=0)
        bias2 = jnp.concatenate([b2a, b2b], axis=1)
        y = jnp.dot(h, w2d, preferred_element_type=jnp.float32) + bias2

        ey = jnp.exp(y)
        acc = g1[:, s:s + 1] * ey[:, :D] + g2[:, s:s + 1] * ey[:, D:]
        acc = jnp.where(acc == 0.0, EPS, acc)
        o_ref[s] = jnp.log(acc).reshape(L, N, D)


@jax.jit
def kernel(x, start_w, start_b, w_gate, W1, b1, W2, b2):
    wgx = jnp.repeat(w_gate.T / N, N, axis=1)
    sbv = (start_b[0] * jnp.sum(w_gate, axis=0)).reshape(E, 1)

    out = pl.pallas_call(
        _body,
        grid=(B // SB,),
        in_specs=[
            pl.BlockSpec((SB, L, N, D), lambda b: (b, 0, 0, 0)),
            pl.BlockSpec((D, 1), lambda b: (0, 0)),
            pl.BlockSpec((E, 1), lambda b: (0, 0)),
            pl.BlockSpec((E, LN), lambda b: (0, 0)),
            pl.BlockSpec((E, D, D_FF), lambda b: (0, 0, 0)),
            pl.BlockSpec((E, 1, D_FF), lambda b: (0, 0, 0)),
            pl.BlockSpec((E, D_FF, D), lambda b: (0, 0, 0)),
            pl.BlockSpec((E, 1, D), lambda b: (0, 0, 0)),
        ],
        out_specs=pl.BlockSpec((SB, L, N, D), lambda b: (b, 0, 0, 0)),
        out_shape=jax.ShapeDtypeStruct((B, L, N, D), jnp.float32),
    )(x, start_w, sbv, wgx, W1, b1.reshape(E, 1, D_FF), W2,
      b2.reshape(E, 1, D))

    return out
